# Initial kernel scaffold; baseline (speedup 1.0000x reference)
import jax
import jax.numpy as jnp
from jax.experimental import pallas as pl


def kernel(x, edge_index, edge_attr, W1, b1, W2, b2, U1, ub1, U2, ub2, ln_g, ln_b):
    raise NotImplementedError("write your pallas kernel here")


# speedup vs baseline: 1.0742x; 1.0742x over previous
import functools

import jax
import jax.numpy as jnp
from jax import lax
from jax.experimental import pallas as pl
from jax.experimental.pallas import tpu as pltpu
from jax.experimental.pallas import tpu_sc as plsc

H = 128
ED = 16
N = 10000
E = 320000

NC = 2
NS = 16
NW = NC * NS
EB = 128
N_PAD = 10112
R_SUB = N_PAD // NS
BLOCKS_PER_W = 80
CHUNK = 2
NCH = BLOCKS_PER_W // CHUNK
E_PAD = NW * BLOCKS_PER_W * EB


def _pq_body(x_ref, w_ref, o_ref):
    o_ref[0] = jnp.dot(x_ref[...], w_ref[0], preferred_element_type=jnp.float32)


def _stage_pq(x_pad, w1ba):
    return pl.pallas_call(
        _pq_body,
        grid=(2,),
        in_specs=[
            pl.BlockSpec((N_PAD, H), lambda i: (0, 0)),
            pl.BlockSpec((1, H, H), lambda i: (i, 0, 0)),
        ],
        out_specs=pl.BlockSpec((1, N_PAD, H), lambda i: (i, 0, 0)),
        out_shape=jax.ShapeDtypeStruct((2, N_PAD, H), jnp.float32),
    )(x_pad, w1ba)


def _r_body(ea_ref, w_ref, b_ref, o_ref):
    o_ref[...] = (
        jnp.dot(ea_ref[...], w_ref[...], preferred_element_type=jnp.float32)
        + b_ref[...]
    )


def _stage_r(ea_pad, w1c, b1):
    eb = 4096
    return pl.pallas_call(
        _r_body,
        grid=(E_PAD // eb,),
        in_specs=[
            pl.BlockSpec((eb, ED), lambda i: (i, 0)),
            pl.BlockSpec((ED, H), lambda i: (0, 0)),
            pl.BlockSpec((1, H), lambda i: (0, 0)),
        ],
        out_specs=pl.BlockSpec((eb, H), lambda i: (i, 0)),
        out_shape=jax.ShapeDtypeStruct((E_PAD, H), jnp.float32),
    )(ea_pad, w1c, b1.reshape(1, H))


def _sc_body(pq_hbm, r_hbm, srcg_hbm, tgt_hbm, v_hbm, out_hbm,
             srcg_v, tgt_v, a_v, b_v, c_v, v_v, acc, sem):
    c = lax.axis_index("c")
    s = lax.axis_index("s")
    w = s * NC + c

    pltpu.sync_copy(v_hbm, v_v)

    def _zrow(i, carry):
        for j in range(H // 16):
            c_v[i, pl.ds(j * 16, 16)] = jnp.zeros((16,), jnp.float32)
        return carry

    lax.fori_loop(0, EB, _zrow, 0)
    r0 = s * R_SUB
    for k in range(R_SUB // EB):
        pltpu.sync_copy(c_v, acc.at[pl.ds(r0 + k * EB, EB)])
    rem = R_SUB - (R_SUB // EB) * EB
    if rem:
        pltpu.sync_copy(c_v.at[pl.ds(0, rem)],
                        acc.at[pl.ds(r0 + (R_SUB // EB) * EB, rem)])
    plsc.subcore_barrier()

    def _chunk(ch, carry):
        pltpu.sync_copy(srcg_hbm.at[w, ch], srcg_v)
        pltpu.sync_copy(tgt_hbm.at[w, ch], tgt_v)
        for bi in range(CHUNK):
            g1 = pltpu.async_copy(pq_hbm.at[srcg_v.at[bi]], a_v, sem)
            g2 = pltpu.async_copy(pq_hbm.at[tgt_v.at[bi]], b_v, sem)
            pltpu.sync_copy(r_hbm.at[w, ch, bi], c_v)
            g1.wait()
            g2.wait()

            for j in range(H // 16):
                sl = pl.ds(j * 16, 16)
                vj = v_v[sl]

                def _row(i, carry2):
                    t = a_v[i, sl] + b_v[i, sl] + c_v[i, sl]
                    c_v[i, sl] = t / (1.0 + jnp.exp(-t)) + vj
                    return carry2

                lax.fori_loop(0, EB, _row, 0)
            pltpu.sync_copy(c_v, acc.at[tgt_v.at[bi]], add=True)
        return carry

    lax.fori_loop(0, NCH, _chunk, 0)
    plsc.subcore_barrier()

    pltpu.sync_copy(acc.at[pl.ds(r0, R_SUB)],
                    out_hbm.at[c, pl.ds(r0, R_SUB)])


def _stage_sc(pq, r, srcg, tgt, v):
    mesh = plsc.VectorSubcoreMesh(core_axis_name="c", subcore_axis_name="s",
                                  num_cores=NC, num_subcores=NS)
    f = pl.kernel(
        _sc_body,
        out_type=jax.ShapeDtypeStruct((NC, N_PAD, H), jnp.float32),
        mesh=mesh,
        scratch_types=[
            pltpu.VMEM((CHUNK, EB), jnp.int32),
            pltpu.VMEM((CHUNK, EB), jnp.int32),
            pltpu.VMEM((EB, H), jnp.float32),
            pltpu.VMEM((EB, H), jnp.float32),
            pltpu.VMEM((EB, H), jnp.float32),
            pltpu.VMEM((H,), jnp.float32),
            pltpu.VMEM_SHARED((N_PAD, H), jnp.float32),
            pltpu.SemaphoreType.DMA,
        ],
    )
    return f(pq, r.reshape(NW, NCH, CHUNK, EB, H), srcg, tgt, v)


def _post_body(p_ref, x_ref, w2_ref, u1a_ref, u1b_ref, ub1_ref,
               u2_ref, ub2_ref, g_ref, bb_ref, o_ref):
    msum = p_ref[0] + p_ref[1]
    agg = jnp.dot(msum, w2_ref[...], preferred_element_type=jnp.float32)
    x = x_ref[...]
    pre = (jnp.dot(x, u1a_ref[...], preferred_element_type=jnp.float32)
           + jnp.dot(agg, u1b_ref[...], preferred_element_type=jnp.float32)
           + ub1_ref[...])
    h = pre * jax.nn.sigmoid(pre)
    upd = jnp.dot(h, u2_ref[...], preferred_element_type=jnp.float32) + ub2_ref[...]
    y = x + upd
    mean = jnp.mean(y, axis=-1, keepdims=True)
    var = jnp.mean((y - mean) ** 2, axis=-1, keepdims=True)
    y = (y - mean) * lax.rsqrt(var + 1e-5)
    o_ref[...] = y * g_ref[...] + bb_ref[...]


def _stage_post(parts, x, W2, U1, ub1, U2, ub2, ln_g, ln_b):
    nb = 2000
    row = lambda a: a.reshape(1, H)
    return pl.pallas_call(
        _post_body,
        grid=(N // nb,),
        in_specs=[
            pl.BlockSpec((NC, nb, H), lambda i: (0, i, 0)),
            pl.BlockSpec((nb, H), lambda i: (i, 0)),
            pl.BlockSpec((H, H), lambda i: (0, 0)),
            pl.BlockSpec((H, H), lambda i: (0, 0)),
            pl.BlockSpec((H, H), lambda i: (0, 0)),
            pl.BlockSpec((1, H), lambda i: (0, 0)),
            pl.BlockSpec((H, H), lambda i: (0, 0)),
            pl.BlockSpec((1, H), lambda i: (0, 0)),
            pl.BlockSpec((1, H), lambda i: (0, 0)),
            pl.BlockSpec((1, H), lambda i: (0, 0)),
        ],
        out_specs=pl.BlockSpec((nb, H), lambda i: (i, 0)),
        out_shape=jax.ShapeDtypeStruct((N, H), jnp.float32),
    )(parts, x, W2, U1[:H], U1[H:], row(ub1), U2, row(ub2),
      row(ln_g), row(ln_b))


def kernel(x, edge_index, edge_attr, W1, b1, W2, b2, U1, ub1, U2, ub2,
           ln_g, ln_b):
    src = edge_index[0].astype(jnp.int32)
    tgt = edge_index[1].astype(jnp.int32)

    pad_e = E_PAD - E
    srcg = jnp.concatenate([src + N_PAD, jnp.full((pad_e,), N_PAD + N, jnp.int32)])
    tgtv = jnp.concatenate([tgt, jnp.full((pad_e,), N, jnp.int32)])
    srcg = srcg.reshape(NW, NCH, CHUNK, EB)
    tgtv = tgtv.reshape(NW, NCH, CHUNK, EB)

    ea_pad = jnp.concatenate(
        [edge_attr, jnp.zeros((pad_e, ED), jnp.float32)])
    x_pad = jnp.concatenate([x, jnp.zeros((N_PAD - N, H), jnp.float32)])

    w1ba = jnp.stack([W1[H:2 * H], W1[:H]])
    pq = _stage_pq(x_pad, w1ba).reshape(2 * N_PAD, H)
    r = _stage_r(ea_pad, W1[2 * H:], b1)
    v = jnp.linalg.solve(W2.T, b2)
    parts = _stage_sc(pq, r, srcg, tgtv, v)
    return _stage_post(parts[:, :N, :], x, W2, U1, ub1, U2, ub2,
                       ln_g, ln_b)


# speedup vs baseline: 2.3178x; 2.1577x over previous
import functools

import jax
import jax.numpy as jnp
from jax import lax
from jax.experimental import pallas as pl
from jax.experimental.pallas import tpu as pltpu
from jax.experimental.pallas import tpu_sc as plsc

H = 128
ED = 16
N = 10000
E = 320000

NC = 2
NS = 16
NW = NC * NS
EB = 128
N_PAD = 10112
R_SUB = N_PAD // NS
BLOCKS_PER_W = 80
CHUNK = 2
NCH = BLOCKS_PER_W // CHUNK
E_PAD = NW * BLOCKS_PER_W * EB


def _pq_body(x_ref, w_ref, o_ref):
    o_ref[0] = jnp.dot(x_ref[...], w_ref[0], preferred_element_type=jnp.float32)


def _stage_pq(x_pad, w1ba):
    return pl.pallas_call(
        _pq_body,
        grid=(2,),
        in_specs=[
            pl.BlockSpec((N_PAD, H), lambda i: (0, 0)),
            pl.BlockSpec((1, H, H), lambda i: (i, 0, 0)),
        ],
        out_specs=pl.BlockSpec((1, N_PAD, H), lambda i: (i, 0, 0)),
        out_shape=jax.ShapeDtypeStruct((2, N_PAD, H), jnp.float32),
    )(x_pad, w1ba)


def _r_body(ea_ref, w_ref, b_ref, o_ref):
    o_ref[...] = (
        jnp.dot(ea_ref[...], w_ref[...], preferred_element_type=jnp.float32)
        + b_ref[...]
    )


def _stage_r(ea_pad, w1c, b1):
    eb = 4096
    return pl.pallas_call(
        _r_body,
        grid=(E_PAD // eb,),
        in_specs=[
            pl.BlockSpec((eb, ED), lambda i: (i, 0)),
            pl.BlockSpec((ED, H), lambda i: (0, 0)),
            pl.BlockSpec((1, H), lambda i: (0, 0)),
        ],
        out_specs=pl.BlockSpec((eb, H), lambda i: (i, 0)),
        out_shape=jax.ShapeDtypeStruct((E_PAD, H), jnp.float32),
    )(ea_pad, w1c, b1.reshape(1, H))


def _sc_body(pq_hbm, r_hbm, srcg_hbm, tgt_hbm, v_hbm, out_hbm,
             srcg_v, tgt_v, a_v, b_v, c_v, v_v, acc, sem_a, sem_b, sem_c):
    c = lax.axis_index("c")
    s = lax.axis_index("s")
    w = s * NC + c

    pltpu.sync_copy(v_hbm, v_v)

    def _zrow(i, carry):
        for j in range(H // 16):
            c_v[i, pl.ds(j * 16, 16)] = jnp.zeros((16,), jnp.float32)
        return carry

    lax.fori_loop(0, EB, _zrow, 0)
    r0 = s * R_SUB
    for k in range(R_SUB // EB):
        pltpu.sync_copy(c_v, acc.at[pl.ds(r0 + k * EB, EB)])
    rem = R_SUB - (R_SUB // EB) * EB
    if rem:
        pltpu.sync_copy(c_v.at[pl.ds(0, rem)],
                        acc.at[pl.ds(r0 + (R_SUB // EB) * EB, rem)])
    plsc.subcore_barrier()

    vjs = [v_v[pl.ds(j * 16, 16)] for j in range(H // 16)]

    def _chunk(ch, carry):
        pltpu.sync_copy(srcg_hbm.at[w, pl.ds(ch * CHUNK, CHUNK)], srcg_v)
        pltpu.sync_copy(tgt_hbm.at[w, pl.ds(ch * CHUNK, CHUNK)], tgt_v)
        for p in range(CHUNK):
            bi = ch * CHUNK + p
            pltpu.async_copy(pq_hbm.at[srcg_v.at[p]], a_v, sem_a)
            pltpu.async_copy(pq_hbm.at[tgt_v.at[p]], b_v, sem_b)
            pltpu.async_copy(r_hbm.at[w, bi], c_v, sem_c)
            pltpu.make_async_copy(pq_hbm.at[srcg_v.at[p]], a_v, sem_a).wait()
            pltpu.make_async_copy(pq_hbm.at[tgt_v.at[p]], b_v, sem_b).wait()
            pltpu.make_async_copy(r_hbm.at[w, bi], c_v, sem_c).wait()

            def _row(i, carry2):
                for j in range(H // 16):
                    sl = pl.ds(j * 16, 16)
                    t = a_v[i, sl] + b_v[i, sl] + c_v[i, sl]
                    c_v[i, sl] = t / (1.0 + jnp.exp(-t)) + vjs[j]
                return carry2

            lax.fori_loop(0, EB, _row, 0)
            pltpu.sync_copy(c_v, acc.at[tgt_v.at[p]], add=True)
        return carry

    lax.fori_loop(0, NCH, _chunk, 0)
    plsc.subcore_barrier()

    pltpu.sync_copy(acc.at[pl.ds(r0, R_SUB)],
                    out_hbm.at[c, pl.ds(r0, R_SUB)])


def _stage_sc(pq, r, srcg, tgt, v):
    mesh = plsc.VectorSubcoreMesh(core_axis_name="c", subcore_axis_name="s",
                                  num_cores=NC, num_subcores=NS)
    f = pl.kernel(
        _sc_body,
        out_type=jax.ShapeDtypeStruct((NC, N_PAD, H), jnp.float32),
        mesh=mesh,
        scratch_types=[
            pltpu.VMEM((CHUNK, EB), jnp.int32),
            pltpu.VMEM((CHUNK, EB), jnp.int32),
            pltpu.VMEM((EB, H), jnp.float32),
            pltpu.VMEM((EB, H), jnp.float32),
            pltpu.VMEM((EB, H), jnp.float32),
            pltpu.VMEM((H,), jnp.float32),
            pltpu.VMEM_SHARED((N_PAD, H), jnp.float32),
            pltpu.SemaphoreType.DMA,
            pltpu.SemaphoreType.DMA,
            pltpu.SemaphoreType.DMA,
        ],
    )
    return f(pq, r.reshape(NW, BLOCKS_PER_W, EB, H), srcg, tgt, v)


def _post_body(p_ref, x_ref, w2_ref, u1a_ref, u1b_ref, ub1_ref,
               u2_ref, ub2_ref, g_ref, bb_ref, o_ref):
    msum = p_ref[0] + p_ref[1]
    agg = jnp.dot(msum, w2_ref[...], preferred_element_type=jnp.float32)
    x = x_ref[...]
    pre = (jnp.dot(x, u1a_ref[...], preferred_element_type=jnp.float32)
           + jnp.dot(agg, u1b_ref[...], preferred_element_type=jnp.float32)
           + ub1_ref[...])
    h = pre * jax.nn.sigmoid(pre)
    upd = jnp.dot(h, u2_ref[...], preferred_element_type=jnp.float32) + ub2_ref[...]
    y = x + upd
    mean = jnp.mean(y, axis=-1, keepdims=True)
    var = jnp.mean((y - mean) ** 2, axis=-1, keepdims=True)
    y = (y - mean) * lax.rsqrt(var + 1e-5)
    o_ref[...] = y * g_ref[...] + bb_ref[...]


def _stage_post(parts, x, W2, U1, ub1, U2, ub2, ln_g, ln_b):
    nb = 2000
    row = lambda a: a.reshape(1, H)
    return pl.pallas_call(
        _post_body,
        grid=(N // nb,),
        in_specs=[
            pl.BlockSpec((NC, nb, H), lambda i: (0, i, 0)),
            pl.BlockSpec((nb, H), lambda i: (i, 0)),
            pl.BlockSpec((H, H), lambda i: (0, 0)),
            pl.BlockSpec((H, H), lambda i: (0, 0)),
            pl.BlockSpec((H, H), lambda i: (0, 0)),
            pl.BlockSpec((1, H), lambda i: (0, 0)),
            pl.BlockSpec((H, H), lambda i: (0, 0)),
            pl.BlockSpec((1, H), lambda i: (0, 0)),
            pl.BlockSpec((1, H), lambda i: (0, 0)),
            pl.BlockSpec((1, H), lambda i: (0, 0)),
        ],
        out_specs=pl.BlockSpec((nb, H), lambda i: (i, 0)),
        out_shape=jax.ShapeDtypeStruct((N, H), jnp.float32),
    )(parts, x, W2, U1[:H], U1[H:], row(ub1), U2, row(ub2),
      row(ln_g), row(ln_b))


def kernel(x, edge_index, edge_attr, W1, b1, W2, b2, U1, ub1, U2, ub2,
           ln_g, ln_b):
    src = edge_index[0].astype(jnp.int32)
    tgt = edge_index[1].astype(jnp.int32)

    pad_e = E_PAD - E
    srcg = jnp.concatenate([src + N_PAD, jnp.full((pad_e,), N_PAD + N, jnp.int32)])
    tgtv = jnp.concatenate([tgt, jnp.full((pad_e,), N, jnp.int32)])
    srcg = srcg.reshape(NW, BLOCKS_PER_W, EB)
    tgtv = tgtv.reshape(NW, BLOCKS_PER_W, EB)

    ea_pad = jnp.concatenate(
        [edge_attr, jnp.zeros((pad_e, ED), jnp.float32)])
    x_pad = jnp.concatenate([x, jnp.zeros((N_PAD - N, H), jnp.float32)])

    w1ba = jnp.stack([W1[H:2 * H], W1[:H]])
    pq = _stage_pq(x_pad, w1ba).reshape(2 * N_PAD, H)
    r = _stage_r(ea_pad, W1[2 * H:], b1)
    v = jnp.linalg.solve(W2.T, b2)
    parts = _stage_sc(pq, r, srcg, tgtv, v)
    return _stage_post(parts[:, :N, :], x, W2, U1, ub1, U2, ub2,
                       ln_g, ln_b)


# speedup vs baseline: 3.9946x; 1.7235x over previous
import functools

import jax
import jax.numpy as jnp
from jax import lax
from jax.experimental import pallas as pl
from jax.experimental.pallas import tpu as pltpu
from jax.experimental.pallas import tpu_sc as plsc

H = 128
ED = 16
N = 10000
E = 320000

NC = 2
NS = 16
NW = NC * NS
EB = 56
N_PAD = 10112
R_SUB = N_PAD // NS
BLOCKS_PER_W = 180
CHUNK = 2
NCH = BLOCKS_PER_W // CHUNK
E_PAD = NW * BLOCKS_PER_W * EB


def _pq_body(x_ref, w_ref, o_ref):
    o_ref[0] = jnp.dot(x_ref[...], w_ref[0], preferred_element_type=jnp.float32)


def _stage_pq(x_pad, w1ba):
    return pl.pallas_call(
        _pq_body,
        grid=(2,),
        in_specs=[
            pl.BlockSpec((N_PAD, H), lambda i: (0, 0)),
            pl.BlockSpec((1, H, H), lambda i: (i, 0, 0)),
        ],
        out_specs=pl.BlockSpec((1, N_PAD, H), lambda i: (i, 0, 0)),
        out_shape=jax.ShapeDtypeStruct((2, N_PAD, H), jnp.float32),
    )(x_pad, w1ba)


def _r_body(ea_ref, w_ref, b_ref, o_ref):
    o_ref[...] = (
        jnp.dot(ea_ref[...], w_ref[...], preferred_element_type=jnp.float32)
        + b_ref[...]
    )


def _stage_r(ea_pad, w1c, b1):
    eb = 3584
    return pl.pallas_call(
        _r_body,
        grid=(E_PAD // eb,),
        in_specs=[
            pl.BlockSpec((eb, ED), lambda i: (i, 0)),
            pl.BlockSpec((ED, H), lambda i: (0, 0)),
            pl.BlockSpec((1, H), lambda i: (0, 0)),
        ],
        out_specs=pl.BlockSpec((eb, H), lambda i: (i, 0)),
        out_shape=jax.ShapeDtypeStruct((E_PAD, H), jnp.float32),
    )(ea_pad, w1c, b1.reshape(1, H))


def _sc_body(pq_hbm, r_hbm, srcg_hbm, tgt_hbm, v_hbm, out_hbm,
             srcg_v, tgt_v, a_v, b_v, c_v, v_v, acc,
             sem0, sem1, sem_i):
    c = lax.axis_index("c")
    s = lax.axis_index("s")
    w = s * NC + c
    sems = (sem0, sem1)
    NB = BLOCKS_PER_W

    pltpu.sync_copy(v_hbm, v_v)

    def _zrow(i, carry):
        for j in range(H // 16):
            c_v[0, i, pl.ds(j * 16, 16)] = jnp.zeros((16,), jnp.float32)
        return carry

    lax.fori_loop(0, EB, _zrow, 0)
    r0 = s * R_SUB
    for k in range(R_SUB // EB):
        pltpu.sync_copy(c_v.at[0], acc.at[pl.ds(r0 + k * EB, EB)])
    rem = R_SUB - (R_SUB // EB) * EB
    if rem:
        pltpu.sync_copy(c_v.at[0, pl.ds(0, rem)],
                        acc.at[pl.ds(r0 + (R_SUB // EB) * EB, rem)])
    plsc.subcore_barrier()

    vjs = [v_v[pl.ds(j * 16, 16)] for j in range(H // 16)]

    pltpu.sync_copy(srcg_hbm.at[w, pl.ds(0, CHUNK)], srcg_v.at[0])
    pltpu.sync_copy(tgt_hbm.at[w, pl.ds(0, CHUNK)], tgt_v.at[0])
    pltpu.async_copy(srcg_hbm.at[w, pl.ds(CHUNK, CHUNK)], srcg_v.at[1], sem_i)
    pltpu.async_copy(tgt_hbm.at[w, pl.ds(CHUNK, CHUNK)], tgt_v.at[1], sem_i)
    for p in range(CHUNK):
        pltpu.async_copy(pq_hbm.at[srcg_v.at[0, p]], a_v.at[p], sems[p])
        pltpu.async_copy(pq_hbm.at[tgt_v.at[0, p]], b_v.at[p], sems[p])
        pltpu.async_copy(r_hbm.at[w, p], c_v.at[p], sems[p])

    def _chunk(ch, carry):
        cur = lax.rem(ch, 3)
        nxt = lax.rem(ch + 1, 3)
        for p in range(CHUNK):
            bi = ch * CHUNK + p
            pltpu.make_async_copy(pq_hbm.at[srcg_v.at[cur, p]], a_v.at[p],
                                  sems[p]).wait()
            pltpu.make_async_copy(pq_hbm.at[tgt_v.at[cur, p]], b_v.at[p],
                                  sems[p]).wait()
            pltpu.make_async_copy(r_hbm.at[w, bi], c_v.at[p], sems[p]).wait()

            def _row(i, carry2):
                for j in range(H // 16):
                    sl = pl.ds(j * 16, 16)
                    t = a_v[p, i, sl] + b_v[p, i, sl] + c_v[p, i, sl]
                    c_v[p, i, sl] = t / (1.0 + jnp.exp(-t)) + vjs[j]
                return carry2

            lax.fori_loop(0, EB, _row, 0)
            pltpu.sync_copy(c_v.at[p], acc.at[tgt_v.at[cur, p]], add=True)

            if p == 0:
                pltpu.make_async_copy(
                    srcg_hbm.at[w, pl.ds(0, CHUNK)], srcg_v.at[nxt],
                    sem_i).wait()
                pltpu.make_async_copy(
                    tgt_hbm.at[w, pl.ds(0, CHUNK)], tgt_v.at[nxt],
                    sem_i).wait()
                ch2 = jnp.minimum(ch + 2, NCH - 1)
                nn = lax.rem(ch + 2, 3)
                pltpu.async_copy(srcg_hbm.at[w, pl.ds(ch2 * CHUNK, CHUNK)],
                                 srcg_v.at[nn], sem_i)
                pltpu.async_copy(tgt_hbm.at[w, pl.ds(ch2 * CHUNK, CHUNK)],
                                 tgt_v.at[nn], sem_i)

            bn = jnp.minimum(bi + CHUNK, NB - 1)
            pltpu.async_copy(pq_hbm.at[srcg_v.at[nxt, p]], a_v.at[p], sems[p])
            pltpu.async_copy(pq_hbm.at[tgt_v.at[nxt, p]], b_v.at[p], sems[p])
            pltpu.async_copy(r_hbm.at[w, bn], c_v.at[p], sems[p])
        return carry

    lax.fori_loop(0, NCH, _chunk, 0)

    for p in range(CHUNK):
        pltpu.make_async_copy(pq_hbm.at[srcg_v.at[0, p]], a_v.at[p],
                              sems[p]).wait()
        pltpu.make_async_copy(pq_hbm.at[tgt_v.at[0, p]], b_v.at[p],
                              sems[p]).wait()
        pltpu.make_async_copy(r_hbm.at[w, p], c_v.at[p], sems[p]).wait()
    pltpu.make_async_copy(srcg_hbm.at[w, pl.ds(0, CHUNK)], srcg_v.at[0],
                          sem_i).wait()
    pltpu.make_async_copy(tgt_hbm.at[w, pl.ds(0, CHUNK)], tgt_v.at[0],
                          sem_i).wait()
    plsc.subcore_barrier()

    pltpu.sync_copy(acc.at[pl.ds(r0, R_SUB)],
                    out_hbm.at[c, pl.ds(r0, R_SUB)])


def _stage_sc(pq, r, srcg, tgt, v):
    mesh = plsc.VectorSubcoreMesh(core_axis_name="c", subcore_axis_name="s",
                                  num_cores=NC, num_subcores=NS)
    f = pl.kernel(
        _sc_body,
        out_type=jax.ShapeDtypeStruct((NC, N_PAD, H), jnp.float32),
        mesh=mesh,
        scratch_types=[
            pltpu.VMEM((3, CHUNK, EB), jnp.int32),
            pltpu.VMEM((3, CHUNK, EB), jnp.int32),
            pltpu.VMEM((2, EB, H), jnp.float32),
            pltpu.VMEM((2, EB, H), jnp.float32),
            pltpu.VMEM((2, EB, H), jnp.float32),
            pltpu.VMEM((H,), jnp.float32),
            pltpu.VMEM_SHARED((N_PAD, H), jnp.float32),
            pltpu.SemaphoreType.DMA,
            pltpu.SemaphoreType.DMA,
            pltpu.SemaphoreType.DMA,
        ],
    )
    return f(pq, r.reshape(NW, BLOCKS_PER_W, EB, H), srcg, tgt, v)


def _post_body(p_ref, x_ref, w2_ref, u1a_ref, u1b_ref, ub1_ref,
               u2_ref, ub2_ref, g_ref, bb_ref, o_ref):
    msum = p_ref[0] + p_ref[1]
    agg = jnp.dot(msum, w2_ref[...], preferred_element_type=jnp.float32)
    x = x_ref[...]
    pre = (jnp.dot(x, u1a_ref[...], preferred_element_type=jnp.float32)
           + jnp.dot(agg, u1b_ref[...], preferred_element_type=jnp.float32)
           + ub1_ref[...])
    h = pre * jax.nn.sigmoid(pre)
    upd = jnp.dot(h, u2_ref[...], preferred_element_type=jnp.float32) + ub2_ref[...]
    y = x + upd
    mean = jnp.mean(y, axis=-1, keepdims=True)
    var = jnp.mean((y - mean) ** 2, axis=-1, keepdims=True)
    y = (y - mean) * lax.rsqrt(var + 1e-5)
    o_ref[...] = y * g_ref[...] + bb_ref[...]


def _stage_post(parts, x, W2, U1, ub1, U2, ub2, ln_g, ln_b):
    nb = 2000
    row = lambda a: a.reshape(1, H)
    return pl.pallas_call(
        _post_body,
        grid=(N // nb,),
        in_specs=[
            pl.BlockSpec((NC, nb, H), lambda i: (0, i, 0)),
            pl.BlockSpec((nb, H), lambda i: (i, 0)),
            pl.BlockSpec((H, H), lambda i: (0, 0)),
            pl.BlockSpec((H, H), lambda i: (0, 0)),
            pl.BlockSpec((H, H), lambda i: (0, 0)),
            pl.BlockSpec((1, H), lambda i: (0, 0)),
            pl.BlockSpec((H, H), lambda i: (0, 0)),
            pl.BlockSpec((1, H), lambda i: (0, 0)),
            pl.BlockSpec((1, H), lambda i: (0, 0)),
            pl.BlockSpec((1, H), lambda i: (0, 0)),
        ],
        out_specs=pl.BlockSpec((nb, H), lambda i: (i, 0)),
        out_shape=jax.ShapeDtypeStruct((N, H), jnp.float32),
    )(parts, x, W2, U1[:H], U1[H:], row(ub1), U2, row(ub2),
      row(ln_g), row(ln_b))


def kernel(x, edge_index, edge_attr, W1, b1, W2, b2, U1, ub1, U2, ub2,
           ln_g, ln_b):
    src = edge_index[0].astype(jnp.int32)
    tgt = edge_index[1].astype(jnp.int32)

    pad_e = E_PAD - E
    srcg = jnp.concatenate([src + N_PAD, jnp.full((pad_e,), N_PAD + N, jnp.int32)])
    tgtv = jnp.concatenate([tgt, jnp.full((pad_e,), N, jnp.int32)])
    srcg = srcg.reshape(NW, BLOCKS_PER_W, EB)
    tgtv = tgtv.reshape(NW, BLOCKS_PER_W, EB)

    ea_pad = jnp.concatenate(
        [edge_attr, jnp.zeros((pad_e, ED), jnp.float32)])
    x_pad = jnp.concatenate([x, jnp.zeros((N_PAD - N, H), jnp.float32)])

    w1ba = jnp.stack([W1[H:2 * H], W1[:H]])
    pq = _stage_pq(x_pad, w1ba).reshape(2 * N_PAD, H)
    r = _stage_r(ea_pad, W1[2 * H:], b1)
    v = jnp.linalg.solve(W2.T, b2)
    parts = _stage_sc(pq, r, srcg, tgtv, v)
    return _stage_post(parts[:, :N, :], x, W2, U1, ub1, U2, ub2,
                       ln_g, ln_b)
